# baseline (speedup 1.0000x reference)
import functools

import jax
import jax.numpy as jnp
from jax import lax
from jax.experimental import pallas as pl
from jax.experimental.pallas import tpu as pltpu
from jax.experimental.pallas import tpu_sc as plsc

_N = 10000
_E = 320000
_D = 128
_NCLS = 40

_NPAD = 10240
_NC = 2
_NS = 16
_NW = _NC * _NS
_B = 256
_K = 40
_EPAD = _NW * _K * _B
_PAD_ROW = _N + 1
_RPT = _NPAD // _NS
_DEGW = 16
_Q = 2
_QW = _D // _Q


_MESH = plsc.VectorSubcoreMesh(core_axis_name="c", subcore_axis_name="s")


@functools.cache
def _make_agg_pass(with_deg: bool):
  def body(src_hbm, dst_hbm, t0, t1, zeros_hbm, deg_init_hbm, ones_hbm,
           *rest):
    if with_deg:
      (out_hbm, deg_out_hbm,
       sidx, didx, rows, acc, gsem, ones_v, deg_acc) = rest
    else:
      out_hbm, sidx, didx, rows, acc, gsem = rest
    cid = lax.axis_index("c")
    sid = lax.axis_index("s")
    wid = cid * _NS + sid
    r0 = sid * _RPT

    pltpu.sync_copy(src_hbm.at[wid], sidx)
    pltpu.sync_copy(dst_hbm.at[wid], didx)

    for q, tbl in enumerate((t0, t1)):
      @pl.when(cid == 0)
      def _():
        pltpu.sync_copy(tbl.at[pl.ds(r0, _RPT)], acc.at[pl.ds(r0, _RPT)])

      @pl.when(cid != 0)
      def _():
        pltpu.sync_copy(zeros_hbm.at[pl.ds(r0, _RPT)],
                        acc.at[pl.ds(r0, _RPT)])

      plsc.subcore_barrier()

      pltpu.async_copy(tbl.at[sidx.at[0]], rows.at[0], gsem)

      @pl.loop(0, _K)
      def _(j):
        buf = lax.rem(j, 2)
        pltpu.make_async_copy(tbl.at[sidx.at[j]], rows.at[buf], gsem).wait()

        @pl.when(j + 1 < _K)
        def _():
          pltpu.async_copy(tbl.at[sidx.at[j + 1]], rows.at[1 - buf], gsem)

        pltpu.sync_copy(rows.at[buf], acc.at[didx.at[j]], add=True)

      plsc.subcore_barrier()

      pltpu.sync_copy(acc.at[pl.ds(r0, _RPT)],
                      out_hbm.at[q].at[cid].at[pl.ds(r0, _RPT)])

    if with_deg:
      pltpu.sync_copy(deg_init_hbm.at[cid].at[pl.ds(r0, _RPT)],
                      deg_acc.at[pl.ds(r0, _RPT)])
      pltpu.sync_copy(ones_hbm, ones_v)

      plsc.subcore_barrier()

      @pl.loop(0, _K)
      def _(j):
        pltpu.sync_copy(ones_v, deg_acc.at[didx.at[j]], add=True)

      plsc.subcore_barrier()

      pltpu.sync_copy(deg_acc.at[pl.ds(r0, _RPT)],
                      deg_out_hbm.at[cid].at[pl.ds(r0, _RPT)])

  out_type = [jax.ShapeDtypeStruct((_Q, _NC, _NPAD, _QW), jnp.float32)]
  scratch_types = [
      pltpu.VMEM((_K, _B), jnp.int32),
      pltpu.VMEM((_K, _B), jnp.int32),
      pltpu.VMEM((2, _B, _QW), jnp.float32),
      pltpu.VMEM_SHARED((_NPAD, _QW), jnp.float32),
      pltpu.SemaphoreType.DMA,
  ]
  if with_deg:
    out_type.append(jax.ShapeDtypeStruct((_NC, _NPAD, _DEGW), jnp.float32))
    scratch_types += [
        pltpu.VMEM((_B, _DEGW), jnp.float32),
        pltpu.VMEM_SHARED((_NPAD, _DEGW), jnp.float32),
    ]

  return pl.kernel(
      body,
      out_type=out_type,
      mesh=_MESH,
      compiler_params=pltpu.CompilerParams(use_tc_tiling_on_sc=False),
      scratch_types=scratch_types)


_TC_R = 1024


def _quarter_out_specs():
  return [pl.BlockSpec((_TC_R, _QW), lambda i: (i, 0)) for _ in range(_Q)]


def _quarter_out_shapes(n):
  return [jax.ShapeDtypeStruct((n, _QW), jnp.float32) for _ in range(_Q)]


def _store_quarters(r, orefs):
  for q, oref in enumerate(orefs):
    oref[...] = r[:, q * _QW:(q + 1) * _QW]


def _mm_body(x_ref, w_ref, *orefs):
  r = jnp.dot(x_ref[...], w_ref[...], preferred_element_type=jnp.float32)
  _store_quarters(r, orefs)


def _mm(x, w):
  n, d = x.shape
  return pl.pallas_call(
      _mm_body,
      grid=(n // _TC_R,),
      in_specs=[
          pl.BlockSpec((_TC_R, d), lambda i: (i, 0)),
          pl.BlockSpec((d, _D), lambda i: (0, 0)),
      ],
      out_specs=_quarter_out_specs(),
      out_shape=_quarter_out_shapes(n),
  )(x, w)


def _combine_h(prefs, d0_ref, d1_ref, b_ref):
  deg = d0_ref[...][:, :1] + d1_ref[...][:, :1]
  agg = jnp.concatenate(
      [prefs[2 * q][...] + prefs[2 * q + 1][...] for q in range(_Q)], axis=1)
  return agg / deg + b_ref[...]


def _comb_mm_body(*refs):
  np_ = 2 * _Q
  prefs, (d0, d1, b_ref, w_ref), orefs = refs[:np_], refs[np_:np_ + 4], refs[np_ + 4:]
  h = jnp.maximum(_combine_h(prefs, d0, d1, b_ref), 0.0)
  r = jnp.dot(h, w_ref[...], preferred_element_type=jnp.float32)
  _store_quarters(r, orefs)


def _partial_in_specs():
  return [pl.BlockSpec((_TC_R, _QW), lambda i: (i, 0)) for _ in range(2 * _Q)]


def _comb_mm(parts, d0, d1, b, w):
  n = d0.shape[0]
  return pl.pallas_call(
      _comb_mm_body,
      grid=(n // _TC_R,),
      in_specs=_partial_in_specs() + [
          pl.BlockSpec((_TC_R, _DEGW), lambda i: (i, 0)),
          pl.BlockSpec((_TC_R, _DEGW), lambda i: (i, 0)),
          pl.BlockSpec((1, _D), lambda i: (0, 0)),
          pl.BlockSpec((_D, _D), lambda i: (0, 0)),
      ],
      out_specs=_quarter_out_specs(),
      out_shape=_quarter_out_shapes(n),
  )(*parts, d0, d1, b, w)


def _final_body(*refs):
  np_ = 2 * _Q
  prefs, (d0, d1, b_ref, o_ref) = refs[:np_], refs[np_:]
  o_ref[...] = _combine_h(prefs, d0, d1, b_ref)


def _final(parts, d0, d1, b):
  n = d0.shape[0]
  return pl.pallas_call(
      _final_body,
      grid=(n // _TC_R,),
      in_specs=_partial_in_specs() + [
          pl.BlockSpec((_TC_R, _DEGW), lambda i: (i, 0)),
          pl.BlockSpec((_TC_R, _DEGW), lambda i: (i, 0)),
          pl.BlockSpec((1, _D), lambda i: (0, 0)),
      ],
      out_specs=pl.BlockSpec((_TC_R, _D), lambda i: (i, 0)),
      out_shape=jax.ShapeDtypeStruct((n, _D), jnp.float32),
  )(*parts, d0, d1, b)


def _split_parts(p):
  return [p[q, c] for q in range(_Q) for c in range(_NC)]


def kernel(x, edge_index, W1, b1, W2, b2, W3, b3):
  src = edge_index[0].astype(jnp.int32)
  dst = edge_index[1].astype(jnp.int32)
  pad = jnp.full((_EPAD - _E,), _PAD_ROW, jnp.int32)
  srcp = jnp.concatenate([src, pad]).reshape(_NW, _K, _B)
  dstp = jnp.concatenate([dst, pad]).reshape(_NW, _K, _B)

  xp = jnp.pad(x, ((0, _NPAD - _N), (0, 0)))
  zeros32 = jnp.zeros((_NPAD, _QW), jnp.float32)
  deg_init = jnp.concatenate(
      [jnp.ones((1, _NPAD, _DEGW), jnp.float32),
       jnp.zeros((1, _NPAD, _DEGW), jnp.float32)])
  ones_b = jnp.ones((_B, _DEGW), jnp.float32)

  agg_deg = _make_agg_pass(True)
  agg = _make_agg_pass(False)

  hw1 = _mm(xp, W1)
  p1, degp = agg_deg(srcp, dstp, *hw1, zeros32, deg_init, ones_b)
  d0, d1 = degp[0], degp[1]
  hw2 = _comb_mm(_split_parts(p1), d0, d1, b1.reshape(1, -1), W2)
  p2 = agg(srcp, dstp, *hw2, zeros32, deg_init, ones_b)[0]
  w3p = jnp.pad(W3, ((0, 0), (0, _D - _NCLS)))
  hw3 = _comb_mm(_split_parts(p2), d0, d1, b2.reshape(1, -1), w3p)
  p3 = agg(srcp, dstp, *hw3, zeros32, deg_init, ones_b)[0]
  b3p = jnp.pad(b3, (0, _D - _NCLS)).reshape(1, -1)
  outp = _final(_split_parts(p3), d0, d1, b3p)
  return outp[:_N, :_NCLS]


# speedup vs baseline: 1.2864x; 1.2864x over previous
import functools

import jax
import jax.numpy as jnp
from jax import lax
from jax.experimental import pallas as pl
from jax.experimental.pallas import tpu as pltpu
from jax.experimental.pallas import tpu_sc as plsc

_N = 10000
_E = 320000
_D = 128
_NCLS = 40

_NPAD = 10240
_NC = 2
_NS = 16
_NW = _NC * _NS
_B = 128
_K = 79
_EPAD = _NW * _K * _B
_PAD_ROW = _N + 1
_RPT = _NPAD // _NS
_DEGW = 16
_Q = 2
_QW = _D // _Q


_MESH = plsc.VectorSubcoreMesh(core_axis_name="c", subcore_axis_name="s")


@functools.cache
def _make_agg_pass(with_deg: bool):
  def body(src_hbm, dst_hbm, t0, t1, zeros_hbm, deg_init_hbm, ones_hbm,
           *rest):
    if with_deg:
      (out_hbm, deg_out_hbm,
       sidx, didx, rows, acc, gsem, ones_v, deg_acc) = rest
    else:
      out_hbm, sidx, didx, rows, acc, gsem = rest
    cid = lax.axis_index("c")
    sid = lax.axis_index("s")
    wid = cid * _NS + sid
    r0 = sid * _RPT

    pltpu.sync_copy(src_hbm.at[wid], sidx)
    pltpu.sync_copy(dst_hbm.at[wid], didx)

    for q, tbl in enumerate((t0, t1)):
      @pl.when(cid == 0)
      def _():
        pltpu.sync_copy(tbl.at[pl.ds(r0, _RPT)], acc.at[pl.ds(r0, _RPT)])

      @pl.when(cid != 0)
      def _():
        pltpu.sync_copy(zeros_hbm.at[pl.ds(r0, _RPT)],
                        acc.at[pl.ds(r0, _RPT)])

      plsc.subcore_barrier()

      pltpu.async_copy(tbl.at[sidx.at[0]], rows.at[0], gsem)

      @pl.loop(0, _K)
      def _(j):
        buf = lax.rem(j, 2)
        pltpu.make_async_copy(tbl.at[sidx.at[j]], rows.at[buf], gsem).wait()

        @pl.when(j + 1 < _K)
        def _():
          pltpu.async_copy(tbl.at[sidx.at[j + 1]], rows.at[1 - buf], gsem)

        pltpu.sync_copy(rows.at[buf], acc.at[didx.at[j]], add=True)

      plsc.subcore_barrier()

      pltpu.sync_copy(acc.at[pl.ds(r0, _RPT)],
                      out_hbm.at[q].at[cid].at[pl.ds(r0, _RPT)])

    if with_deg:
      pltpu.sync_copy(deg_init_hbm.at[cid].at[pl.ds(r0, _RPT)],
                      deg_acc.at[pl.ds(r0, _RPT)])
      pltpu.sync_copy(ones_hbm, ones_v)

      plsc.subcore_barrier()

      @pl.loop(0, _K)
      def _(j):
        pltpu.sync_copy(ones_v, deg_acc.at[didx.at[j]], add=True)

      plsc.subcore_barrier()

      pltpu.sync_copy(deg_acc.at[pl.ds(r0, _RPT)],
                      deg_out_hbm.at[cid].at[pl.ds(r0, _RPT)])

  out_type = [jax.ShapeDtypeStruct((_Q, _NC, _NPAD, _QW), jnp.float32)]
  scratch_types = [
      pltpu.VMEM((_K, _B), jnp.int32),
      pltpu.VMEM((_K, _B), jnp.int32),
      pltpu.VMEM((2, _B, _QW), jnp.float32),
      pltpu.VMEM_SHARED((_NPAD, _QW), jnp.float32),
      pltpu.SemaphoreType.DMA,
  ]
  if with_deg:
    out_type.append(jax.ShapeDtypeStruct((_NC, _NPAD, _DEGW), jnp.float32))
    scratch_types += [
        pltpu.VMEM((_B, _DEGW), jnp.float32),
        pltpu.VMEM_SHARED((_NPAD, _DEGW), jnp.float32),
    ]

  return pl.kernel(
      body,
      out_type=out_type,
      mesh=_MESH,
      compiler_params=pltpu.CompilerParams(use_tc_tiling_on_sc=False),
      scratch_types=scratch_types)


_TC_R = 1024


def _quarter_out_specs():
  return [pl.BlockSpec((_TC_R, _QW), lambda i: (i, 0)) for _ in range(_Q)]


def _quarter_out_shapes(n):
  return [jax.ShapeDtypeStruct((n, _QW), jnp.float32) for _ in range(_Q)]


def _store_quarters(r, orefs):
  for q, oref in enumerate(orefs):
    oref[...] = r[:, q * _QW:(q + 1) * _QW]


def _mm_body(x_ref, w_ref, *orefs):
  r = jnp.dot(x_ref[...], w_ref[...], preferred_element_type=jnp.float32)
  _store_quarters(r, orefs)


def _mm(x, w):
  n, d = x.shape
  return pl.pallas_call(
      _mm_body,
      grid=(n // _TC_R,),
      in_specs=[
          pl.BlockSpec((_TC_R, d), lambda i: (i, 0)),
          pl.BlockSpec((d, _D), lambda i: (0, 0)),
      ],
      out_specs=_quarter_out_specs(),
      out_shape=_quarter_out_shapes(n),
  )(x, w)


def _combine_h(prefs, d0_ref, d1_ref, b_ref):
  deg = d0_ref[...][:, :1] + d1_ref[...][:, :1]
  agg = jnp.concatenate(
      [prefs[2 * q][...] + prefs[2 * q + 1][...] for q in range(_Q)], axis=1)
  return agg / deg + b_ref[...]


def _comb_mm_body(*refs):
  np_ = 2 * _Q
  prefs, (d0, d1, b_ref, w_ref), orefs = refs[:np_], refs[np_:np_ + 4], refs[np_ + 4:]
  h = jnp.maximum(_combine_h(prefs, d0, d1, b_ref), 0.0)
  r = jnp.dot(h, w_ref[...], preferred_element_type=jnp.float32)
  _store_quarters(r, orefs)


def _partial_in_specs():
  return [pl.BlockSpec((_TC_R, _QW), lambda i: (i, 0)) for _ in range(2 * _Q)]


def _comb_mm(parts, d0, d1, b, w):
  n = d0.shape[0]
  return pl.pallas_call(
      _comb_mm_body,
      grid=(n // _TC_R,),
      in_specs=_partial_in_specs() + [
          pl.BlockSpec((_TC_R, _DEGW), lambda i: (i, 0)),
          pl.BlockSpec((_TC_R, _DEGW), lambda i: (i, 0)),
          pl.BlockSpec((1, _D), lambda i: (0, 0)),
          pl.BlockSpec((_D, _D), lambda i: (0, 0)),
      ],
      out_specs=_quarter_out_specs(),
      out_shape=_quarter_out_shapes(n),
  )(*parts, d0, d1, b, w)


def _final_body(*refs):
  np_ = 2 * _Q
  prefs, (d0, d1, b_ref, o_ref) = refs[:np_], refs[np_:]
  o_ref[...] = _combine_h(prefs, d0, d1, b_ref)


def _final(parts, d0, d1, b):
  n = d0.shape[0]
  return pl.pallas_call(
      _final_body,
      grid=(n // _TC_R,),
      in_specs=_partial_in_specs() + [
          pl.BlockSpec((_TC_R, _DEGW), lambda i: (i, 0)),
          pl.BlockSpec((_TC_R, _DEGW), lambda i: (i, 0)),
          pl.BlockSpec((1, _D), lambda i: (0, 0)),
      ],
      out_specs=pl.BlockSpec((_TC_R, _D), lambda i: (i, 0)),
      out_shape=jax.ShapeDtypeStruct((n, _D), jnp.float32),
  )(*parts, d0, d1, b)


def _split_parts(p):
  return [p[q, c] for q in range(_Q) for c in range(_NC)]


def kernel(x, edge_index, W1, b1, W2, b2, W3, b3):
  src = edge_index[0].astype(jnp.int32)
  dst = edge_index[1].astype(jnp.int32)
  pad = jnp.full((_EPAD - _E,), _PAD_ROW, jnp.int32)
  srcp = jnp.concatenate([src, pad]).reshape(_NW, _K, _B)
  dstp = jnp.concatenate([dst, pad]).reshape(_NW, _K, _B)

  xp = jnp.pad(x, ((0, _NPAD - _N), (0, 0)))
  zeros32 = jnp.zeros((_NPAD, _QW), jnp.float32)
  deg_init = jnp.concatenate(
      [jnp.ones((1, _NPAD, _DEGW), jnp.float32),
       jnp.zeros((1, _NPAD, _DEGW), jnp.float32)])
  ones_b = jnp.ones((_B, _DEGW), jnp.float32)

  agg_deg = _make_agg_pass(True)
  agg = _make_agg_pass(False)

  hw1 = _mm(xp, W1)
  p1, degp = agg_deg(srcp, dstp, *hw1, zeros32, deg_init, ones_b)
  d0, d1 = degp[0], degp[1]
  hw2 = _comb_mm(_split_parts(p1), d0, d1, b1.reshape(1, -1), W2)
  p2 = agg(srcp, dstp, *hw2, zeros32, deg_init, ones_b)[0]
  w3p = jnp.pad(W3, ((0, 0), (0, _D - _NCLS)))
  hw3 = _comb_mm(_split_parts(p2), d0, d1, b2.reshape(1, -1), w3p)
  p3 = agg(srcp, dstp, *hw3, zeros32, deg_init, ones_b)[0]
  b3p = jnp.pad(b3, (0, _D - _NCLS)).reshape(1, -1)
  outp = _final(_split_parts(p3), d0, d1, b3p)
  return outp[:_N, :_NCLS]
